# baseline (speedup 1.0000x reference)
import dataclasses
import functools

import jax
import jax.numpy as jnp
from jax import lax
from jax.experimental import pallas as pl
from jax.experimental.pallas import tpu as pltpu
from jax.experimental.pallas import tpu_sc as plsc

IMG_TOKEN = 151655

NC = 2
NS = 16
NW = NC * NS


def _fix_body(n_img_rows, ids_ref, fix_ref):
    ids = ids_ref[...]
    mask = ids == IMG_TOKEN
    x = mask.astype(jnp.int32)
    rows, cols = x.shape
    k = 1
    while k < cols:
        x = x + jnp.concatenate(
            [jnp.zeros((rows, k), jnp.int32), x[:, : cols - k]], axis=1)
        k *= 2
    row_tot = x[:, cols - 1 : cols]
    t = row_tot
    k = 1
    while k < rows:
        t = t + jnp.concatenate(
            [jnp.zeros((k, 1), jnp.int32), t[: rows - k, :]], axis=0)
        k *= 2
    cs = x + (t - row_tot)
    img_idx = jnp.clip(cs - 1, 0, n_img_rows - 1)
    fix_ref[...] = jnp.where(mask, img_idx, -1)


def _compute_fix(ids_flat, n_img_rows):
    n = ids_flat.shape[0]
    cols = 128
    rows = n // cols
    ids2d = ids_flat.reshape(rows, cols)
    fix = pl.pallas_call(
        functools.partial(_fix_body, n_img_rows),
        out_shape=jax.ShapeDtypeStruct((rows, cols), jnp.int32),
    )(ids2d)
    return fix.reshape(-1)


def _sc_gather(ids_flat, fix_flat, table, image):
    n = ids_flat.shape[0]
    d = table.shape[1]
    per_w = n // NW
    chunk = 16
    mesh = plsc.VectorSubcoreMesh(core_axis_name="c", subcore_axis_name="s")
    cp = pltpu.CompilerParams()
    if "needs_layout_passes" in pltpu.CompilerParams.__dataclass_fields__:
        cp = dataclasses.replace(cp, needs_layout_passes=False)

    @functools.partial(
        pl.kernel,
        out_type=jax.ShapeDtypeStruct((n, d), jnp.float32),
        mesh=mesh,
        compiler_params=cp,
        scratch_types=[
            pltpu.VMEM((per_w,), jnp.int32),
            pltpu.VMEM((per_w,), jnp.int32),
            pltpu.VMEM((chunk, d), jnp.float32),
            pltpu.SemaphoreType.DMA,
        ],
    )
    def k(ids_hbm, fix_hbm, table_hbm, img_hbm, out_hbm,
          idx_v, fix_v, rows_v, sem):
        wid = lax.axis_index("s") * NC + lax.axis_index("c")
        base = wid * per_w
        pltpu.sync_copy(ids_hbm.at[pl.ds(base, per_w)], idx_v)
        pltpu.sync_copy(fix_hbm.at[pl.ds(base, per_w)], fix_v)

        @pl.loop(0, per_w, step=chunk)
        def _(c0):
            pltpu.async_copy(
                table_hbm.at[idx_v.at[pl.ds(c0, chunk)]], rows_v, sem).wait()

            fvec = fix_v[pl.ds(c0, chunk)]
            cmax = jnp.max(fvec)

            @pl.when(cmax >= 0)
            def _():
                lanes = lax.iota(jnp.int32, chunk)
                for j in range(chunk):
                    fj = jnp.max(jnp.where(lanes == j, fvec, -1))

                    @pl.when(fj >= 0)
                    def _(fj=fj, j=j):
                        pltpu.sync_copy(
                            img_hbm.at[pl.ds(fj, 1)], rows_v.at[pl.ds(j, 1)])

            pltpu.sync_copy(rows_v, out_hbm.at[pl.ds(base + c0, chunk)])

    return k(ids_flat, fix_flat, table, image)


def kernel(input_ids, image_embeds, embed_weight):
    b, s = input_ids.shape
    d = embed_weight.shape[1]
    ids_flat = input_ids.reshape(-1)
    fix_flat = _compute_fix(ids_flat, image_embeds.shape[0])
    out = _sc_gather(ids_flat, fix_flat, embed_weight, image_embeds)
    return out.reshape(b, s, d)


# speedup vs baseline: 1.1253x; 1.1253x over previous
import dataclasses
import functools

import jax
import jax.numpy as jnp
from jax import lax
from jax.experimental import pallas as pl
from jax.experimental.pallas import tpu as pltpu
from jax.experimental.pallas import tpu_sc as plsc

IMG_TOKEN = 151655

NC = 2
NS = 16
NW = NC * NS


def _fix_body(n_img_rows, ids_ref, fix_ref):
    ids = ids_ref[...]
    mask = ids == IMG_TOKEN
    x = mask.astype(jnp.int32)
    rows, cols = x.shape
    k = 1
    while k < cols:
        x = x + jnp.concatenate(
            [jnp.zeros((rows, k), jnp.int32), x[:, : cols - k]], axis=1)
        k *= 2
    row_tot = x[:, cols - 1 : cols]
    t = row_tot
    k = 1
    while k < rows:
        t = t + jnp.concatenate(
            [jnp.zeros((k, 1), jnp.int32), t[: rows - k, :]], axis=0)
        k *= 2
    cs = x + (t - row_tot)
    img_idx = jnp.clip(cs - 1, 0, n_img_rows - 1)
    fix_ref[...] = jnp.where(mask, img_idx, -1)


def _compute_fix(ids_flat, n_img_rows):
    n = ids_flat.shape[0]
    cols = 128
    rows = n // cols
    ids2d = ids_flat.reshape(rows, cols)
    fix = pl.pallas_call(
        functools.partial(_fix_body, n_img_rows),
        out_shape=jax.ShapeDtypeStruct((rows, cols), jnp.int32),
    )(ids2d)
    return fix.reshape(-1)


def _sc_gather(ids_flat, fix_flat, table, image):
    n = ids_flat.shape[0]
    d = table.shape[1]
    per_w = n // NW
    chunk = 16
    mesh = plsc.VectorSubcoreMesh(core_axis_name="c", subcore_axis_name="s")
    cp = pltpu.CompilerParams()
    if "needs_layout_passes" in pltpu.CompilerParams.__dataclass_fields__:
        cp = dataclasses.replace(cp, needs_layout_passes=False)

    @functools.partial(
        pl.kernel,
        out_type=jax.ShapeDtypeStruct((n, d), jnp.float32),
        mesh=mesh,
        compiler_params=cp,
        scratch_types=[
            pltpu.VMEM((per_w,), jnp.int32),
            pltpu.VMEM((per_w,), jnp.int32),
            pltpu.VMEM((chunk, d), jnp.float32),
            pltpu.VMEM((chunk, d), jnp.float32),
            pltpu.SemaphoreType.DMA,
            pltpu.SemaphoreType.DMA,
            pltpu.SemaphoreType.DMA,
            pltpu.SemaphoreType.DMA,
        ],
    )
    def k(ids_hbm, fix_hbm, table_hbm, img_hbm, out_hbm,
          idx_v, fix_v, rows0, rows1, gsem0, gsem1, wsem0, wsem1):
        wid = lax.axis_index("s") * NC + lax.axis_index("c")
        base = wid * per_w
        pltpu.sync_copy(ids_hbm.at[pl.ds(base, per_w)], idx_v)
        pltpu.sync_copy(fix_hbm.at[pl.ds(base, per_w)], fix_v)
        bufs = ((rows0, gsem0, wsem0), (rows1, gsem1, wsem1))

        def gather_start(c, buf, gs):
            pltpu.async_copy(table_hbm.at[idx_v.at[pl.ds(c, chunk)]], buf, gs)

        def gather_wait(buf, gs):
            pltpu.make_async_copy(
                table_hbm.at[idx_v.at[pl.ds(0, chunk)]], buf, gs).wait()

        def write_wait(buf, ws):
            pltpu.make_async_copy(buf, out_hbm.at[pl.ds(0, chunk)], ws).wait()

        def fixup(c, buf):
            fvec = fix_v[pl.ds(c, chunk)]
            cmax = jnp.max(fvec)

            @pl.when(cmax >= 0)
            def _():
                lanes = lax.iota(jnp.int32, chunk)
                for j in range(chunk):
                    fj = jnp.max(jnp.where(lanes == j, fvec, -1))

                    @pl.when(fj >= 0)
                    def _(fj=fj, j=j):
                        pltpu.sync_copy(
                            img_hbm.at[pl.ds(fj, 1)], buf.at[pl.ds(j, 1)])

        gather_start(0, rows0, gsem0)
        gather_start(chunk, rows1, gsem1)

        @pl.loop(0, per_w, step=2 * chunk)
        def _(c0):
            for b, (buf, gs, ws) in enumerate(bufs):
                c = c0 + b * chunk
                gather_wait(buf, gs)
                fixup(c, buf)
                pltpu.async_copy(buf, out_hbm.at[pl.ds(base + c, chunk)], ws)
                nxt = c + 2 * chunk

                @pl.when(nxt < per_w)
                def _(buf=buf, gs=gs, ws=ws, nxt=nxt):
                    write_wait(buf, ws)
                    gather_start(nxt, buf, gs)

        write_wait(rows0, wsem0)
        write_wait(rows1, wsem1)

    return k(ids_flat, fix_flat, table, image)


def kernel(input_ids, image_embeds, embed_weight):
    b, s = input_ids.shape
    d = embed_weight.shape[1]
    ids_flat = input_ids.reshape(-1)
    fix_flat = _compute_fix(ids_flat, image_embeds.shape[0])
    out = _sc_gather(ids_flat, fix_flat, embed_weight, image_embeds)
    return out.reshape(b, s, d)


# speedup vs baseline: 1.6790x; 1.4920x over previous
import dataclasses
import functools

import jax
import jax.numpy as jnp
from jax import lax
from jax.experimental import pallas as pl
from jax.experimental.pallas import tpu as pltpu
from jax.experimental.pallas import tpu_sc as plsc

IMG_TOKEN = 151655

NC = 2
NS = 16
NW = NC * NS


def _fix_body(n_img_rows, ids_ref, fix_ref):
    ids = ids_ref[...]
    mask = ids == IMG_TOKEN
    x = mask.astype(jnp.int32)
    rows, cols = x.shape
    k = 1
    while k < cols:
        x = x + jnp.concatenate(
            [jnp.zeros((rows, k), jnp.int32), x[:, : cols - k]], axis=1)
        k *= 2
    row_tot = x[:, cols - 1 : cols]
    t = row_tot
    k = 1
    while k < rows:
        t = t + jnp.concatenate(
            [jnp.zeros((k, 1), jnp.int32), t[: rows - k, :]], axis=0)
        k *= 2
    cs = x + (t - row_tot)
    img_idx = jnp.clip(cs - 1, 0, n_img_rows - 1)
    fix_ref[...] = jnp.where(mask, img_idx, -1)


def _compute_fix(ids_flat, n_img_rows):
    n = ids_flat.shape[0]
    cols = 128
    rows = n // cols
    ids2d = ids_flat.reshape(rows, cols)
    fix = pl.pallas_call(
        functools.partial(_fix_body, n_img_rows),
        out_shape=jax.ShapeDtypeStruct((rows, cols), jnp.int32),
    )(ids2d)
    return fix.reshape(-1)


def _sc_gather(ids_flat, fix_flat, table, image):
    n = ids_flat.shape[0]
    d = table.shape[1]
    per_w = n // NW
    chunk = 16
    mesh = plsc.VectorSubcoreMesh(core_axis_name="c", subcore_axis_name="s")
    cp = pltpu.CompilerParams()
    if "needs_layout_passes" in pltpu.CompilerParams.__dataclass_fields__:
        cp = dataclasses.replace(cp, needs_layout_passes=False)

    @functools.partial(
        pl.kernel,
        out_type=jax.ShapeDtypeStruct((n, d), jnp.float32),
        mesh=mesh,
        compiler_params=cp,
        scratch_types=[
            pltpu.VMEM((per_w,), jnp.int32),
            pltpu.VMEM((per_w,), jnp.int32),
            pltpu.VMEM((chunk, d), jnp.float32),
            pltpu.VMEM((chunk, d), jnp.float32),
            pltpu.SemaphoreType.DMA,
            pltpu.SemaphoreType.DMA,
            pltpu.SemaphoreType.DMA,
            pltpu.SemaphoreType.DMA,
            pltpu.SemaphoreType.DMA,
        ],
    )
    def k(ids_hbm, fix_hbm, table_hbm, img_hbm, out_hbm,
          idx_v, fix_v, rows0, rows1, gsem0, gsem1, wsem0, wsem1, fsem):
        wid = lax.axis_index("s") * NC + lax.axis_index("c")
        base = wid * per_w
        pltpu.sync_copy(ids_hbm.at[pl.ds(base, per_w)], idx_v)
        pltpu.sync_copy(fix_hbm.at[pl.ds(base, per_w)], fix_v)
        bufs = ((rows0, gsem0, wsem0), (rows1, gsem1, wsem1))

        def gather_start(c, buf, gs):
            pltpu.async_copy(table_hbm.at[idx_v.at[pl.ds(c, chunk)]], buf, gs)

        def gather_wait(buf, gs):
            pltpu.make_async_copy(
                table_hbm.at[idx_v.at[pl.ds(0, chunk)]], buf, gs).wait()

        def write_wait(buf, ws):
            pltpu.make_async_copy(buf, out_hbm.at[pl.ds(0, chunk)], ws).wait()

        def fixup(c, buf):
            fvec = fix_v[pl.ds(c, chunk)]
            cmax = jnp.max(fvec)

            @pl.when(cmax >= 0)
            def _():
                lanes = lax.iota(jnp.int32, chunk)
                for j in range(chunk):
                    fj = jnp.max(jnp.where(lanes == j, fvec, -1))

                    @pl.when(fj >= 0)
                    def _(fj=fj, j=j):
                        pltpu.async_copy(
                            img_hbm.at[pl.ds(fj, 1)], buf.at[pl.ds(j, 1)],
                            fsem)
                for j in range(chunk):
                    fj = jnp.max(jnp.where(lanes == j, fvec, -1))

                    @pl.when(fj >= 0)
                    def _(j=j):
                        pltpu.make_async_copy(
                            img_hbm.at[pl.ds(0, 1)], buf.at[pl.ds(j, 1)],
                            fsem).wait()

        gather_start(0, rows0, gsem0)
        gather_start(chunk, rows1, gsem1)

        @pl.loop(0, per_w, step=2 * chunk)
        def _(c0):
            for b, (buf, gs, ws) in enumerate(bufs):
                c = c0 + b * chunk
                gather_wait(buf, gs)
                fixup(c, buf)
                pltpu.async_copy(buf, out_hbm.at[pl.ds(base + c, chunk)], ws)
                nxt = c + 2 * chunk

                @pl.when(nxt < per_w)
                def _(buf=buf, gs=gs, ws=ws, nxt=nxt):
                    write_wait(buf, ws)
                    gather_start(nxt, buf, gs)

        write_wait(rows0, wsem0)
        write_wait(rows1, wsem1)

    return k(ids_flat, fix_flat, table, image)


def kernel(input_ids, image_embeds, embed_weight):
    b, s = input_ids.shape
    d = embed_weight.shape[1]
    ids_flat = input_ids.reshape(-1)
    fix_flat = _compute_fix(ids_flat, image_embeds.shape[0])
    out = _sc_gather(ids_flat, fix_flat, embed_weight, image_embeds)
    return out.reshape(b, s, d)


# speedup vs baseline: 1.6887x; 1.0058x over previous
import dataclasses
import functools

import jax
import jax.numpy as jnp
from jax import lax
from jax.experimental import pallas as pl
from jax.experimental.pallas import tpu as pltpu
from jax.experimental.pallas import tpu_sc as plsc

IMG_TOKEN = 151655

NC = 2
NS = 16
NW = NC * NS


def _fix_body(n_img_rows, ids_ref, fix_ref):
    ids = ids_ref[...]
    mask = ids == IMG_TOKEN
    x = mask.astype(jnp.int32)
    rows, cols = x.shape
    k = 1
    while k < cols:
        x = x + jnp.concatenate(
            [jnp.zeros((rows, k), jnp.int32), x[:, : cols - k]], axis=1)
        k *= 2
    row_tot = x[:, cols - 1 : cols]
    t = row_tot
    k = 1
    while k < rows:
        t = t + jnp.concatenate(
            [jnp.zeros((k, 1), jnp.int32), t[: rows - k, :]], axis=0)
        k *= 2
    cs = x + (t - row_tot)
    img_idx = jnp.clip(cs - 1, 0, n_img_rows - 1)
    fix_ref[...] = jnp.where(mask, img_idx, -1)


def _compute_fix(ids_flat, n_img_rows):
    n = ids_flat.shape[0]
    cols = 128
    rows = n // cols
    ids2d = ids_flat.reshape(rows, cols)
    fix = pl.pallas_call(
        functools.partial(_fix_body, n_img_rows),
        out_shape=jax.ShapeDtypeStruct((rows, cols), jnp.int32),
    )(ids2d)
    return fix.reshape(-1)


def _sc_gather(ids_flat, fix_flat, table, image):
    n = ids_flat.shape[0]
    d = table.shape[1]
    per_w = n // NW
    chunk = 16
    mesh = plsc.VectorSubcoreMesh(core_axis_name="c", subcore_axis_name="s")
    cp = pltpu.CompilerParams()
    if "needs_layout_passes" in pltpu.CompilerParams.__dataclass_fields__:
        cp = dataclasses.replace(cp, needs_layout_passes=False)

    @functools.partial(
        pl.kernel,
        out_type=jax.ShapeDtypeStruct((n, d), jnp.float32),
        mesh=mesh,
        compiler_params=cp,
        scratch_types=[
            pltpu.VMEM((per_w,), jnp.int32),
            pltpu.VMEM((per_w,), jnp.int32),
            pltpu.VMEM((chunk, d), jnp.float32),
            pltpu.VMEM((chunk, d), jnp.float32),
            pltpu.VMEM((chunk, d), jnp.float32),
            pltpu.SemaphoreType.DMA,
            pltpu.SemaphoreType.DMA,
            pltpu.SemaphoreType.DMA,
            pltpu.SemaphoreType.DMA,
            pltpu.SemaphoreType.DMA,
            pltpu.SemaphoreType.DMA,
            pltpu.SemaphoreType.DMA,
        ],
    )
    def k(ids_hbm, fix_hbm, table_hbm, img_hbm, out_hbm,
          idx_v, fix_v, rows0, rows1, rows2,
          gsem0, gsem1, gsem2, wsem0, wsem1, wsem2, fsem):
        wid = lax.axis_index("s") * NC + lax.axis_index("c")
        base = wid * per_w
        pltpu.sync_copy(ids_hbm.at[pl.ds(base, per_w)], idx_v)
        pltpu.sync_copy(fix_hbm.at[pl.ds(base, per_w)], fix_v)
        bufs = ((rows0, gsem0, wsem0), (rows1, gsem1, wsem1),
                (rows2, gsem2, wsem2))

        def gather_start(c, buf, gs):
            pltpu.async_copy(table_hbm.at[idx_v.at[pl.ds(c, chunk)]], buf, gs)

        def gather_wait(buf, gs):
            pltpu.make_async_copy(
                table_hbm.at[idx_v.at[pl.ds(0, chunk)]], buf, gs).wait()

        def write_wait(buf, ws):
            pltpu.make_async_copy(buf, out_hbm.at[pl.ds(0, chunk)], ws).wait()

        def fixup(c, buf):
            fvec = fix_v[pl.ds(c, chunk)]
            cmax = jnp.max(fvec)

            @pl.when(cmax >= 0)
            def _():
                lanes = lax.iota(jnp.int32, chunk)
                for j in range(chunk):
                    fj = jnp.max(jnp.where(lanes == j, fvec, -1))

                    @pl.when(fj >= 0)
                    def _(fj=fj, j=j):
                        pltpu.async_copy(
                            img_hbm.at[pl.ds(fj, 1)], buf.at[pl.ds(j, 1)],
                            fsem)
                for j in range(chunk):
                    fj = jnp.max(jnp.where(lanes == j, fvec, -1))

                    @pl.when(fj >= 0)
                    def _(j=j):
                        pltpu.make_async_copy(
                            img_hbm.at[pl.ds(0, 1)], buf.at[pl.ds(j, 1)],
                            fsem).wait()

        gather_start(0, rows0, gsem0)
        gather_start(chunk, rows1, gsem1)
        gather_start(2 * chunk, rows2, gsem2)

        def step(c, r):
            buf, gs, ws = bufs[r]
            nbuf_, ngs, nws = bufs[(r + 2) % 3]
            gather_wait(buf, gs)
            fixup(c, buf)
            pltpu.async_copy(buf, out_hbm.at[pl.ds(base + c, chunk)], ws)
            nxt = c + 2 * chunk

            @pl.when((nxt < per_w) & (nxt >= 3 * chunk))
            def _():
                write_wait(nbuf_, nws)
                gather_start(nxt, nbuf_, ngs)

        n_macro = (per_w // chunk) // 3 * 3

        @pl.loop(0, n_macro * chunk, step=3 * chunk)
        def _(c0):
            for r in range(3):
                step(c0 + r * chunk, r)

        for i in range(n_macro, per_w // chunk):
            step(i * chunk, i % 3)

        write_wait(rows0, wsem0)
        write_wait(rows1, wsem1)
        write_wait(rows2, wsem2)

    return k(ids_flat, fix_flat, table, image)


def kernel(input_ids, image_embeds, embed_weight):
    b, s = input_ids.shape
    d = embed_weight.shape[1]
    ids_flat = input_ids.reshape(-1)
    fix_flat = _compute_fix(ids_flat, image_embeds.shape[0])
    out = _sc_gather(ids_flat, fix_flat, embed_weight, image_embeds)
    return out.reshape(b, s, d)
